# baseline (speedup 1.0000x reference)
import functools

import jax
import jax.numpy as jnp
from jax import lax
from jax.experimental import pallas as pl
from jax.experimental.pallas import tpu as pltpu
from jax.experimental.pallas import tpu_sc as plsc

VOCAB = 64
EMBED_DIM = 16

NC = 2
NS = 16
NW = NC * NS

LANES = 16
CH = 512
GROUPS = CH // LANES
NSLOT = 2
ROW_W = VOCAB


def _table_body(e_ref, w_ref, b_ref, o_ref):
    o_ref[...] = lax.dot_general(
        e_ref[...], w_ref[...], (((1,), (1,)), ((), ())),
        preferred_element_type=jnp.float32,
    ) + b_ref[...]


def _fused_table(embed_table, W, b2d):
    return pl.pallas_call(
        _table_body,
        out_shape=jax.ShapeDtypeStruct((VOCAB, VOCAB), jnp.float32),
    )(embed_table, W, b2d)


def _make_sc_expand(n_chunks):
    mesh = plsc.VectorSubcoreMesh(core_axis_name="c", subcore_axis_name="s")
    assert n_chunks % NSLOT == 0

    @functools.partial(
        pl.kernel,
        mesh=mesh,
        compiler_params=pltpu.CompilerParams(needs_layout_passes=False),
        out_type=jax.ShapeDtypeStruct((NW * n_chunks * CH * ROW_W,),
                                      jnp.float32),
        scratch_types=[
            pltpu.VMEM((VOCAB * VOCAB,), jnp.float32),
            pltpu.VMEM((NSLOT, CH), jnp.int32),
            pltpu.VMEM((NSLOT * CH * ROW_W,), jnp.float32),
            pltpu.SemaphoreType.DMA((NSLOT,)),
            pltpu.SemaphoreType.DMA((NSLOT,)),
            pltpu.SemaphoreType.DMA,
        ],
    )
    def sc_expand(table_hbm, ids_hbm, out_hbm, table_v, ids_v, rows_v,
                  sem_ids, sem_out, sem_t):
        w = lax.axis_index("s") * NC + lax.axis_index("c")

        pltpu.async_copy(table_hbm, table_v, sem_t).wait()

        lane_iota = lax.iota(jnp.int32, LANES)
        lane_off = lane_iota * ROW_W

        def fire_ids(c, s):
            pltpu.async_copy(ids_hbm.at[w, c], ids_v.at[s], sem_ids.at[s])

        def wait_ids(c, s):
            pltpu.make_async_copy(ids_hbm.at[w, c], ids_v.at[s],
                                  sem_ids.at[s]).wait()

        def rows_slot(s):
            return rows_v.at[pl.ds(s * CH * ROW_W, CH * ROW_W)]

        def out_slot(c):
            return out_hbm.at[
                pl.ds((w * n_chunks + c) * (CH * ROW_W), CH * ROW_W)]

        def wait_out(c, s):
            pltpu.make_async_copy(rows_slot(s), out_slot(c),
                                  sem_out.at[s]).wait()

        for s in range(NSLOT):
            fire_ids(s, s)

        def body(g, carry):
            for s in range(NSLOT):
                c = g * NSLOT + s

                @pl.when(g > 0)
                def _():
                    wait_out(c - NSLOT, s)

                wait_ids(c, s)

                slot_base = s * CH * ROW_W

                qoff = [lane_iota + q for q in range(0, ROW_W, LANES)]
                bidx = [jnp.full((LANES,), k, jnp.int32)
                        for k in range(LANES)]

                @plsc.parallel_loop(0, GROUPS, step=1)
                def _(t):
                    idv = ids_v[s, pl.ds(t * LANES, LANES)]
                    rbase = idv * ROW_W
                    gbase = slot_base + t * (LANES * ROW_W)
                    for k in range(LANES):
                        rb = jnp.take(rbase, bidx[k])
                        ob = gbase + k * ROW_W
                        for qi, qv in enumerate(qoff):
                            vals = plsc.load_gather(table_v, [rb + qv])
                            rows_v[pl.ds(ob + qi * LANES, LANES)] = vals

                @pl.when(g < n_chunks // NSLOT - 1)
                def _():
                    fire_ids(c + NSLOT, s)

                pltpu.async_copy(rows_slot(s), out_slot(c),
                                 sem_out.at[s])
            return carry

        lax.fori_loop(0, n_chunks // NSLOT, body, 0)
        for s in range(NSLOT):
            wait_out(n_chunks - NSLOT + s, s)

    return sc_expand


def kernel(input_ids, embed_table, W, b):
    B, T = input_ids.shape
    n = B * T
    assert n % (NW * CH) == 0
    n_chunks = n // (NW * CH)

    table = _fused_table(embed_table, W, b.reshape(1, VOCAB))
    ids = input_ids.reshape(NW, n_chunks, CH).astype(jnp.int32)
    out = _make_sc_expand(n_chunks)(table.reshape(VOCAB * VOCAB), ids)
    return out.reshape(B, T, VOCAB)


# speedup vs baseline: 1.2633x; 1.2633x over previous
import functools

import jax
import jax.numpy as jnp
from jax import lax
from jax.experimental import pallas as pl
from jax.experimental.pallas import tpu as pltpu
from jax.experimental.pallas import tpu_sc as plsc

VOCAB = 64
EMBED_DIM = 16

NC = 2
NS = 16
NW = NC * NS

LANES = 16
CH_ROWS = 2
NSLOT = 2
ROW_W = VOCAB


def _table_body(e_ref, w_ref, b_ref, o_ref):
    o_ref[...] = lax.dot_general(
        e_ref[...], w_ref[...], (((1,), (1,)), ((), ())),
        preferred_element_type=jnp.float32,
    ) + b_ref[...]


def _fused_table(embed_table, W, b2d):
    return pl.pallas_call(
        _table_body,
        out_shape=jax.ShapeDtypeStruct((VOCAB, VOCAB), jnp.float32),
    )(embed_table, W, b2d)


def _make_sc_expand(n_chunks, B, T):
    mesh = plsc.VectorSubcoreMesh(core_axis_name="c", subcore_axis_name="s")
    assert n_chunks % NSLOT == 0
    CH = CH_ROWS * T
    CHP = 512
    FULLG = T // LANES
    TAIL = T - FULLG * LANES
    assert CH <= CHP - LANES

    @functools.partial(
        pl.kernel,
        mesh=mesh,
        compiler_params=pltpu.CompilerParams(needs_layout_passes=False),
        out_type=jax.ShapeDtypeStruct((B, T, ROW_W), jnp.float32),
        scratch_types=[
            pltpu.VMEM((VOCAB * VOCAB,), jnp.float32),
            pltpu.VMEM((NSLOT, CHP), jnp.int32),
            pltpu.VMEM((NSLOT * CH_ROWS, T, ROW_W), jnp.float32),
            pltpu.SemaphoreType.DMA((NSLOT,)),
            pltpu.SemaphoreType.DMA((NSLOT,)),
            pltpu.SemaphoreType.DMA,
        ],
    )
    def sc_expand(table_hbm, ids_hbm, out_hbm, table_v, ids_v, rows_v,
                  sem_ids, sem_out, sem_t):
        w = lax.axis_index("s") * NC + lax.axis_index("c")

        pltpu.async_copy(table_hbm, table_v, sem_t).wait()

        lane_iota = lax.iota(jnp.int32, LANES)

        def fire_ids(c, s):
            pltpu.async_copy(ids_hbm.at[w, c], ids_v.at[s], sem_ids.at[s])

        def wait_ids(c, s):
            pltpu.make_async_copy(ids_hbm.at[w, c], ids_v.at[s],
                                  sem_ids.at[s]).wait()

        def rows_slot(s):
            return rows_v.at[pl.ds(s * CH_ROWS, CH_ROWS)]

        def out_slot(c):
            start = pl.multiple_of((w * n_chunks + c) * CH_ROWS, CH_ROWS)
            return out_hbm.at[pl.ds(start, CH_ROWS)]

        def wait_out(c, s):
            pltpu.make_async_copy(rows_slot(s), out_slot(c),
                                  sem_out.at[s]).wait()

        for s in range(NSLOT):
            fire_ids(s, s)

        def body(g, carry):
            for s in range(NSLOT):
                c = g * NSLOT + s

                @pl.when(g > 0)
                def _():
                    wait_out(c - NSLOT, s)

                wait_ids(c, s)

                qoff = [lane_iota + q for q in range(0, ROW_W, LANES)]
                bidx = [jnp.full((LANES,), k, jnp.int32)
                        for k in range(LANES)]

                def expand_group(row, tok0, nk):
                    idv = ids_v[s, pl.ds(tok0, LANES)]
                    rbase = (idv & (VOCAB - 1)) * ROW_W
                    for k in range(nk):
                        rb = jnp.take(rbase, bidx[k])
                        trow = tok0 - (row - s * CH_ROWS) * T + k
                        for qi, qv in enumerate(qoff):
                            vals = plsc.load_gather(table_v, [rb + qv])
                            rows_v[row, trow, pl.ds(qi * LANES, LANES)] = vals

                for r in range(CH_ROWS):
                    row = s * CH_ROWS + r

                    @plsc.parallel_loop(0, FULLG, step=1)
                    def _(tg):
                        expand_group(row, r * T + tg * LANES, LANES)

                    if TAIL:
                        expand_group(row, r * T + FULLG * LANES, TAIL)

                @pl.when(g < n_chunks // NSLOT - 1)
                def _():
                    fire_ids(c + NSLOT, s)

                pltpu.async_copy(rows_slot(s), out_slot(c),
                                 sem_out.at[s])
            return carry

        lax.fori_loop(0, n_chunks // NSLOT, body, 0)
        for s in range(NSLOT):
            wait_out(n_chunks - NSLOT + s, s)

    return sc_expand


def kernel(input_ids, embed_table, W, b):
    B, T = input_ids.shape
    n = B * T
    ch = CH_ROWS * T
    assert n % (NW * ch) == 0
    n_chunks = n // (NW * ch)

    table = _fused_table(embed_table, W, b.reshape(1, VOCAB))
    ids = input_ids.reshape(NW, n_chunks, ch).astype(jnp.int32)
    ids = jnp.pad(ids, ((0, 0), (0, 0), (0, 512 - ch)))
    return _make_sc_expand(n_chunks, B, T)(table.reshape(VOCAB * VOCAB), ids)
